# baseline (speedup 1.0000x reference)
import jax
import jax.numpy as jnp
from jax.experimental import pallas as pl
from jax.experimental.pallas import tpu as pltpu

_BB = 16
_BL = 2048


def _body(x_ref, w_ref, b_ref, mve_ref, pe_ref, o_ref):
    v3 = x_ref[...][:, :, None]
    w = w_ref[0, :]
    bpe = b_ref[0, :][None, :] + pe_ref[...]
    mpe = mve_ref[0, :][None, :] + pe_ref[...]
    xe = v3 * w[None, None, :] + bpe[None, :, :]
    o_ref[...] = jnp.where(v3 == -10.0, mpe[None, :, :], xe)


def kernel(x, W, b, masked_value_embedding, pe):
    B, L, _ = x.shape
    E = pe.shape[1]
    x2 = x.reshape(B, L)
    w2 = W.reshape(1, E)
    b2 = b.reshape(1, E)
    m2 = masked_value_embedding.reshape(1, E)

    out = pl.pallas_call(
        _body,
        grid=(B // _BB, L // _BL),
        in_specs=[
            pl.BlockSpec((_BB, _BL), lambda i, j: (i, j)),
            pl.BlockSpec((1, E), lambda i, j: (0, 0)),
            pl.BlockSpec((1, E), lambda i, j: (0, 0)),
            pl.BlockSpec((1, E), lambda i, j: (0, 0)),
            pl.BlockSpec((_BL, E), lambda i, j: (j, 0)),
        ],
        out_specs=pl.BlockSpec((_BB, _BL, E), lambda i, j: (i, j, 0)),
        out_shape=jax.ShapeDtypeStruct((B, L, E), jnp.float32),
        compiler_params=pltpu.CompilerParams(
            dimension_semantics=("parallel", "parallel"),
        ),
    )(x2, w2, b2, m2, pe)
    return out[None]
